# baseline (speedup 1.0000x reference)
import functools

import jax
import jax.numpy as jnp
from jax import lax
from jax.experimental import pallas as pl
from jax.experimental.pallas import tpu as pltpu
from jax.experimental.pallas import tpu_sc as plsc

Q = 1024
D = 256
K = 65536
TK = 32
CH = 128
NCH = K // CH
BN = 1024
CPB = BN // CH
GRID = K // BN
CAP = 128
NEG = -3.0e38
NSLICE = 4
QS = Q // NSLICE


def _k1a_body(x_ref, c_ref, sims_ref, m3_ref):
    s = lax.dot_general(
        x_ref[...], c_ref[...], (((1,), (1,)), ((), ())),
        preferred_element_type=jnp.float32,
    )
    cms = []
    for c in range(CPB):
        sc_ = s[:, c * CH:(c + 1) * CH]
        sims_ref[:, c, :] = sc_
        cms.append(jnp.max(sc_, axis=1, keepdims=True))
    m3_ref[0] = jnp.concatenate(cms, axis=1)


def _k1a(x, centers):
    q = x.shape[0]
    return pl.pallas_call(
        _k1a_body,
        grid=(GRID,),
        in_specs=[
            pl.BlockSpec((q, D), lambda i: (0, 0)),
            pl.BlockSpec((BN, D), lambda i: (i, 0)),
        ],
        out_specs=[
            pl.BlockSpec((q, CPB, CH), lambda i: (0, i, 0)),
            pl.BlockSpec((1, q, CPB), lambda i: (i, 0, 0)),
        ],
        out_shape=[
            jax.ShapeDtypeStruct((q, NCH, CH), jnp.float32),
            jax.ShapeDtypeStruct((GRID, q, CPB), jnp.float32),
        ],
    )(x, centers)


def _k1c_body(q, m_ref, cids_ref, vals_ref, ms_ref):
    ms_ref[...] = m_ref[...]
    lane = lax.broadcasted_iota(jnp.int32, (q, NCH), 1)
    col = lax.broadcasted_iota(jnp.int32, (q, TK), 1)

    def body(t, carry):
        cv, vv = carry
        m = ms_ref[...]
        mx = jnp.max(m, axis=1, keepdims=True)
        arg = jnp.min(jnp.where(m == mx, lane, NCH), axis=1, keepdims=True)
        cv = jnp.where(col == t, arg, cv)
        vv = jnp.where(col == t, mx, vv)
        ms_ref[...] = jnp.where(lane == arg, NEG, m)
        return cv, vv

    cv0 = jnp.zeros((q, TK), jnp.int32)
    vv0 = jnp.zeros((q, TK), jnp.float32)
    cv, vv = lax.fori_loop(0, TK, body, (cv0, vv0))
    cids_ref[...] = cv
    vals_ref[...] = vv


def _k1c(m):
    q = m.shape[0]
    return pl.pallas_call(
        functools.partial(_k1c_body, q),
        out_shape=[
            jax.ShapeDtypeStruct((q, TK), jnp.int32),
            jax.ShapeDtypeStruct((q, TK), jnp.float32),
        ],
        scratch_shapes=[pltpu.VMEM((q, NCH), jnp.float32)],
    )(m)


NC = 2
NS = 16
NW = NC * NS
NVB = CAP // 16


def _splat(v):
    return jnp.full((16,), v, jnp.int32)


def _k2_body(rpw, sims_ref, cids_ref, vals_ref, cent_ref, x_ref, out_ref,
             cids_v, vals_v, gidxa_v, gidxb_v, chunka_v, chunkb_v,
             bufv_v, bufp_v, tmpv_v, tmpp_v, winv_v, winp_v, winw_v,
             jidx_v, cbuf_v, xbuf_v, obuf_v, sema, semb, semc):
    wid = lax.axis_index("s") * NC + lax.axis_index("c")
    base = wid * rpw
    pltpu.sync_copy(cids_ref.at[pl.ds(base, rpw)], cids_v)
    pltpu.sync_copy(vals_ref.at[pl.ds(base, rpw)], vals_v)
    pltpu.sync_copy(x_ref.at[pl.ds(base, rpw)], xbuf_v)

    iota = lax.iota(jnp.int32, 16)
    lane0 = iota == 0

    def stage_gather(r, gidx_ref):
        rr = jnp.minimum(r, rpw - 1)
        rs = _splat(rr)
        off = _splat((base + rr) * NCH)
        gidx_ref[pl.ds(0, 16)] = plsc.load_gather(cids_v, [rs, iota]) + off
        gidx_ref[pl.ds(16, 16)] = plsc.load_gather(cids_v, [rs, iota + 16]) + off

    def process_row(r, chunk_ref):
        rs = _splat(r)
        t_spl = plsc.load_gather(vals_v, [rs, _splat(TK - 1)])
        m_spl = plsc.load_gather(vals_v, [rs, _splat(0)])

        for j in range(NVB):
            bufv_v[pl.ds(16 * j, 16)] = jnp.full((16,), NEG, jnp.float32)

        def filt_body(c, off):
            cs_ = _splat(c)
            for p in range(CH // 16):
                v = plsc.load_gather(chunk_ref, [cs_, iota + 16 * p])
                msk = v >= t_spl
                cs = plsc.cumsum(jnp.where(msk, 1, 0).astype(jnp.int32))
                pos = off + cs - 1
                ok = msk & (pos < CAP)
                gpos = cs_ * CH + (iota + 16 * p)
                plsc.store_scatter(bufv_v, [pos], v, mask=ok)
                plsc.store_scatter(bufp_v, [pos], gpos, mask=ok)
                off = off + plsc.all_reduce_population_count(msk)
            return off

        offf = lax.fori_loop(0, TK, filt_body, _splat(0))
        surv = jnp.minimum(lax.reduce_max(offf, (0,)), CAP)
        nv = (surv + 15) // 16

        def ext_body(t, _):
            def tb(j, carry):
                val, pay = carry
                jj = j * 16 + iota
                v = plsc.load_gather(bufv_v, [jj])
                p = plsc.load_gather(bufp_v, [jj])
                better = v > val
                val = jnp.where(better, v, val)
                pay = jnp.where(better, jj * 4096 + p, pay)
                return val, pay

            val0 = jnp.full((16,), NEG, jnp.float32)
            val, pay = lax.fori_loop(0, nv, tb, (val0, _splat(0)))
            sv, sp = plsc.sort_key_val(val, pay, descending=True)
            tmpv_v[pl.ds(0, 16)] = sv
            tmpp_v[pl.ds(0, 16)] = sp
            mxv = plsc.load_gather(tmpv_v, [_splat(0)])
            mxp = plsc.load_gather(tmpp_v, [_splat(0)])
            slot = lax.shift_right_logical(mxp, 12)
            gpos = mxp & 4095
            ts = _splat(t)
            plsc.store_scatter(winv_v, [ts], mxv, mask=lane0)
            plsc.store_scatter(winp_v, [ts], gpos, mask=lane0)
            plsc.store_scatter(bufv_v, [slot], jnp.full((16,), NEG, jnp.float32),
                               mask=lane0)
            return 0

        lax.fori_loop(0, TK, ext_body, 0)

        wv0 = jnp.exp(winv_v[pl.ds(0, 16)] - m_spl)
        wv1 = jnp.exp(winv_v[pl.ds(16, 16)] - m_spl)
        csum = plsc.cumsum(wv0 + wv1)
        tmpv_v[pl.ds(0, 16)] = csum
        ssum = plsc.load_gather(tmpv_v, [_splat(15)])
        scale = jnp.full((16,), 0.5, jnp.float32) / ssum
        winw_v[pl.ds(0, 16)] = wv0 * scale
        winw_v[pl.ds(16, 16)] = wv1 * scale

        for h in range(2):
            gp = winp_v[pl.ds(16 * h, 16)]
            cid = plsc.load_gather(cids_v, [rs, lax.shift_right_logical(gp, 7)])
            jidx_v[pl.ds(16 * h, 16)] = cid * CH + (gp & (CH - 1))
        pltpu.async_copy(cent_ref.at[jidx_v], cbuf_v, semc).wait()

        acc = [plsc.load_gather(xbuf_v, [rs, iota + 16 * d])
               for d in range(D // 16)]

        def acc_body(t, acc):
            ts16 = _splat(t)
            wt = plsc.load_gather(winw_v, [ts16])
            new = []
            for d in range(D // 16):
                c = plsc.load_gather(cbuf_v, [ts16, iota + 16 * d])
                new.append(acc[d] + wt * c)
            return tuple(new)

        acc = lax.fori_loop(0, TK, acc_body, tuple(acc))
        for d in range(D // 16):
            plsc.store_scatter(obuf_v, [rs, iota + 16 * d], acc[d])

    stage_gather(0, gidxa_v)
    pltpu.async_copy(sims_ref.at[gidxa_v], chunka_v, sema)

    def pair_body(i, _):
        r0 = 2 * i
        stage_gather(r0 + 1, gidxb_v)
        pltpu.async_copy(sims_ref.at[gidxb_v], chunkb_v, semb)
        pltpu.make_async_copy(sims_ref.at[gidxa_v], chunka_v, sema).wait()
        process_row(r0, chunka_v)
        stage_gather(r0 + 2, gidxa_v)
        pltpu.async_copy(sims_ref.at[gidxa_v], chunka_v, sema)
        pltpu.make_async_copy(sims_ref.at[gidxb_v], chunkb_v, semb).wait()
        process_row(r0 + 1, chunkb_v)
        return 0

    lax.fori_loop(0, rpw // 2, pair_body, 0)
    pltpu.make_async_copy(sims_ref.at[gidxa_v], chunka_v, sema).wait()
    pltpu.sync_copy(obuf_v, out_ref.at[pl.ds(base, rpw)])


def _k2(sims2d, cids, vals, centers, x):
    q = x.shape[0]
    rpw = q // NW
    mesh = plsc.VectorSubcoreMesh(core_axis_name="c", subcore_axis_name="s")
    f = functools.partial(
        pl.kernel,
        out_type=jax.ShapeDtypeStruct((q, D), jnp.float32),
        mesh=mesh,
        compiler_params=pltpu.CompilerParams(needs_layout_passes=False),
        scratch_types=[
            pltpu.VMEM((rpw, TK), jnp.int32),
            pltpu.VMEM((rpw, TK), jnp.float32),
            pltpu.VMEM((TK,), jnp.int32),
            pltpu.VMEM((TK,), jnp.int32),
            pltpu.VMEM((TK, CH), jnp.float32),
            pltpu.VMEM((TK, CH), jnp.float32),
            pltpu.VMEM((CAP,), jnp.float32),
            pltpu.VMEM((CAP,), jnp.int32),
            pltpu.VMEM((16,), jnp.float32),
            pltpu.VMEM((16,), jnp.int32),
            pltpu.VMEM((TK,), jnp.float32),
            pltpu.VMEM((TK,), jnp.int32),
            pltpu.VMEM((TK,), jnp.float32),
            pltpu.VMEM((TK,), jnp.int32),
            pltpu.VMEM((TK, D), jnp.float32),
            pltpu.VMEM((rpw, D), jnp.float32),
            pltpu.VMEM((rpw, D), jnp.float32),
            pltpu.SemaphoreType.DMA,
            pltpu.SemaphoreType.DMA,
            pltpu.SemaphoreType.DMA,
        ],
    )(functools.partial(_k2_body, rpw))
    return f(sims2d, cids, vals, centers, x)


def kernel(x, cluster_centers):
    outs = []
    for s in range(NSLICE):
        xs = lax.slice_in_dim(x, s * QS, (s + 1) * QS, axis=0)
        sims3, m3 = _k1a(xs, cluster_centers)
        m = m3.transpose(1, 0, 2).reshape(QS, NCH)
        cids, vals = _k1c(m)
        sims2d = sims3.reshape(QS * NCH, CH)
        outs.append(_k2(sims2d, cids, vals, cluster_centers, xs))
    return jnp.concatenate(outs, axis=0)


# speedup vs baseline: 1.1794x; 1.1794x over previous
import functools

import jax
import jax.numpy as jnp
from jax import lax
from jax.experimental import pallas as pl
from jax.experimental.pallas import tpu as pltpu
from jax.experimental.pallas import tpu_sc as plsc

Q = 1024
D = 256
K = 65536
TK = 32
CH = 128
NCH = K // CH
BN = 1024
CPB = BN // CH
GRID = K // BN
CAP = 128
NEG = -3.0e38
NSLICE = 2
QS = Q // NSLICE


def _k1a_body(x_ref, c_ref, sims_ref, m3_ref):
    s = lax.dot_general(
        x_ref[...], c_ref[...], (((1,), (1,)), ((), ())),
        preferred_element_type=jnp.float32,
    )
    cms = []
    for c in range(CPB):
        sc_ = s[:, c * CH:(c + 1) * CH]
        sims_ref[:, c, :] = sc_
        cms.append(jnp.max(sc_, axis=1, keepdims=True))
    m3_ref[0] = jnp.concatenate(cms, axis=1)


def _k1a(x, centers):
    q = x.shape[0]
    return pl.pallas_call(
        _k1a_body,
        grid=(GRID,),
        in_specs=[
            pl.BlockSpec((q, D), lambda i: (0, 0)),
            pl.BlockSpec((BN, D), lambda i: (i, 0)),
        ],
        out_specs=[
            pl.BlockSpec((q, CPB, CH), lambda i: (0, i, 0)),
            pl.BlockSpec((1, q, CPB), lambda i: (i, 0, 0)),
        ],
        out_shape=[
            jax.ShapeDtypeStruct((q, NCH, CH), jnp.float32),
            jax.ShapeDtypeStruct((GRID, q, CPB), jnp.float32),
        ],
    )(x, centers)


def _k1c_body(q, m_ref, cids_ref, vals_ref, ms_ref):
    ms_ref[...] = m_ref[...]
    lane = lax.broadcasted_iota(jnp.int32, (q, NCH), 1)
    col = lax.broadcasted_iota(jnp.int32, (q, TK), 1)

    def body(t, carry):
        cv, vv = carry
        m = ms_ref[...]
        mx = jnp.max(m, axis=1, keepdims=True)
        arg = jnp.min(jnp.where(m == mx, lane, NCH), axis=1, keepdims=True)
        cv = jnp.where(col == t, arg, cv)
        vv = jnp.where(col == t, mx, vv)
        ms_ref[...] = jnp.where(lane == arg, NEG, m)
        return cv, vv

    cv0 = jnp.zeros((q, TK), jnp.int32)
    vv0 = jnp.zeros((q, TK), jnp.float32)
    cv, vv = lax.fori_loop(0, TK, body, (cv0, vv0))
    cids_ref[...] = cv
    vals_ref[...] = vv


def _k1c(m):
    q = m.shape[0]
    return pl.pallas_call(
        functools.partial(_k1c_body, q),
        out_shape=[
            jax.ShapeDtypeStruct((q, TK), jnp.int32),
            jax.ShapeDtypeStruct((q, TK), jnp.float32),
        ],
        scratch_shapes=[pltpu.VMEM((q, NCH), jnp.float32)],
    )(m)


NC = 2
NS = 16
NW = NC * NS
NVB = CAP // 16


def _splat(v):
    return jnp.full((16,), v, jnp.int32)


def _k2_body(rpw, sims_ref, cids_ref, vals_ref, cent_ref, x_ref, out_ref,
             cids_v, vals_v, gidxa_v, gidxb_v, chunka_v, chunkb_v,
             bufv_v, bufp_v, tmpv_v, tmpp_v, winv_v, winp_v, winw_v,
             jidx_v, cbuf_v, xbuf_v, obuf_v, sema, semb, semc):
    wid = lax.axis_index("s") * NC + lax.axis_index("c")
    base = wid * rpw
    pltpu.sync_copy(cids_ref.at[pl.ds(base, rpw)], cids_v)
    pltpu.sync_copy(vals_ref.at[pl.ds(base, rpw)], vals_v)
    pltpu.sync_copy(x_ref.at[pl.ds(base, rpw)], xbuf_v)

    iota = lax.iota(jnp.int32, 16)
    lane0 = iota == 0

    def stage_gather(r, gidx_ref):
        rr = jnp.minimum(r, rpw - 1)
        rs = _splat(rr)
        off = _splat((base + rr) * NCH)
        gidx_ref[pl.ds(0, 16)] = plsc.load_gather(cids_v, [rs, iota]) + off
        gidx_ref[pl.ds(16, 16)] = plsc.load_gather(cids_v, [rs, iota + 16]) + off

    def process_row(r, chunk_ref):
        rs = _splat(r)
        t_spl = plsc.load_gather(vals_v, [rs, _splat(TK - 1)])
        m_spl = plsc.load_gather(vals_v, [rs, _splat(0)])

        for j in range(NVB):
            bufv_v[pl.ds(16 * j, 16)] = jnp.full((16,), NEG, jnp.float32)

        def filt_body(c, off):
            cs_ = _splat(c)
            for p in range(CH // 16):
                v = plsc.load_gather(chunk_ref, [cs_, iota + 16 * p])
                msk = v >= t_spl
                cs = plsc.cumsum(jnp.where(msk, 1, 0).astype(jnp.int32))
                pos = off + cs - 1
                ok = msk & (pos < CAP)
                gpos = cs_ * CH + (iota + 16 * p)
                plsc.store_scatter(bufv_v, [pos], v, mask=ok)
                plsc.store_scatter(bufp_v, [pos], gpos, mask=ok)
                off = off + plsc.all_reduce_population_count(msk)
            return off

        offf = lax.fori_loop(0, TK, filt_body, _splat(0))
        surv = jnp.minimum(lax.reduce_max(offf, (0,)), CAP)
        nv = (surv + 15) // 16

        def ext_body(t, _):
            def tb(j, carry):
                val, pay = carry
                jj = j * 16 + iota
                v = plsc.load_gather(bufv_v, [jj])
                p = plsc.load_gather(bufp_v, [jj])
                better = v > val
                val = jnp.where(better, v, val)
                pay = jnp.where(better, jj * 4096 + p, pay)
                return val, pay

            val0 = jnp.full((16,), NEG, jnp.float32)
            val, pay = lax.fori_loop(0, nv, tb, (val0, _splat(0)))
            sv, sp = plsc.sort_key_val(val, pay, descending=True)
            tmpv_v[pl.ds(0, 16)] = sv
            tmpp_v[pl.ds(0, 16)] = sp
            mxv = plsc.load_gather(tmpv_v, [_splat(0)])
            mxp = plsc.load_gather(tmpp_v, [_splat(0)])
            slot = lax.shift_right_logical(mxp, 12)
            gpos = mxp & 4095
            ts = _splat(t)
            plsc.store_scatter(winv_v, [ts], mxv, mask=lane0)
            plsc.store_scatter(winp_v, [ts], gpos, mask=lane0)
            plsc.store_scatter(bufv_v, [slot], jnp.full((16,), NEG, jnp.float32),
                               mask=lane0)
            return 0

        lax.fori_loop(0, TK, ext_body, 0)

        wv0 = jnp.exp(winv_v[pl.ds(0, 16)] - m_spl)
        wv1 = jnp.exp(winv_v[pl.ds(16, 16)] - m_spl)
        csum = plsc.cumsum(wv0 + wv1)
        tmpv_v[pl.ds(0, 16)] = csum
        ssum = plsc.load_gather(tmpv_v, [_splat(15)])
        scale = jnp.full((16,), 0.5, jnp.float32) / ssum
        winw_v[pl.ds(0, 16)] = wv0 * scale
        winw_v[pl.ds(16, 16)] = wv1 * scale

        for h in range(2):
            gp = winp_v[pl.ds(16 * h, 16)]
            cid = plsc.load_gather(cids_v, [rs, lax.shift_right_logical(gp, 7)])
            jidx_v[pl.ds(16 * h, 16)] = cid * CH + (gp & (CH - 1))
        pltpu.async_copy(cent_ref.at[jidx_v], cbuf_v, semc).wait()

        acc = [plsc.load_gather(xbuf_v, [rs, iota + 16 * d])
               for d in range(D // 16)]

        def acc_body(t, acc):
            ts16 = _splat(t)
            wt = plsc.load_gather(winw_v, [ts16])
            new = []
            for d in range(D // 16):
                c = plsc.load_gather(cbuf_v, [ts16, iota + 16 * d])
                new.append(acc[d] + wt * c)
            return tuple(new)

        acc = lax.fori_loop(0, TK, acc_body, tuple(acc))
        for d in range(D // 16):
            plsc.store_scatter(obuf_v, [rs, iota + 16 * d], acc[d])

    stage_gather(0, gidxa_v)
    pltpu.async_copy(sims_ref.at[gidxa_v], chunka_v, sema)

    def pair_body(i, _):
        r0 = 2 * i
        stage_gather(r0 + 1, gidxb_v)
        pltpu.async_copy(sims_ref.at[gidxb_v], chunkb_v, semb)
        pltpu.make_async_copy(sims_ref.at[gidxa_v], chunka_v, sema).wait()
        process_row(r0, chunka_v)
        stage_gather(r0 + 2, gidxa_v)
        pltpu.async_copy(sims_ref.at[gidxa_v], chunka_v, sema)
        pltpu.make_async_copy(sims_ref.at[gidxb_v], chunkb_v, semb).wait()
        process_row(r0 + 1, chunkb_v)
        return 0

    lax.fori_loop(0, rpw // 2, pair_body, 0)
    pltpu.make_async_copy(sims_ref.at[gidxa_v], chunka_v, sema).wait()
    pltpu.sync_copy(obuf_v, out_ref.at[pl.ds(base, rpw)])


def _k2(sims2d, cids, vals, centers, x):
    q = x.shape[0]
    rpw = q // NW
    mesh = plsc.VectorSubcoreMesh(core_axis_name="c", subcore_axis_name="s")
    f = functools.partial(
        pl.kernel,
        out_type=jax.ShapeDtypeStruct((q, D), jnp.float32),
        mesh=mesh,
        compiler_params=pltpu.CompilerParams(needs_layout_passes=False),
        scratch_types=[
            pltpu.VMEM((rpw, TK), jnp.int32),
            pltpu.VMEM((rpw, TK), jnp.float32),
            pltpu.VMEM((TK,), jnp.int32),
            pltpu.VMEM((TK,), jnp.int32),
            pltpu.VMEM((TK, CH), jnp.float32),
            pltpu.VMEM((TK, CH), jnp.float32),
            pltpu.VMEM((CAP,), jnp.float32),
            pltpu.VMEM((CAP,), jnp.int32),
            pltpu.VMEM((16,), jnp.float32),
            pltpu.VMEM((16,), jnp.int32),
            pltpu.VMEM((TK,), jnp.float32),
            pltpu.VMEM((TK,), jnp.int32),
            pltpu.VMEM((TK,), jnp.float32),
            pltpu.VMEM((TK,), jnp.int32),
            pltpu.VMEM((TK, D), jnp.float32),
            pltpu.VMEM((rpw, D), jnp.float32),
            pltpu.VMEM((rpw, D), jnp.float32),
            pltpu.SemaphoreType.DMA,
            pltpu.SemaphoreType.DMA,
            pltpu.SemaphoreType.DMA,
        ],
    )(functools.partial(_k2_body, rpw))
    return f(sims2d, cids, vals, centers, x)


def kernel(x, cluster_centers):
    outs = []
    for s in range(NSLICE):
        xs = lax.slice_in_dim(x, s * QS, (s + 1) * QS, axis=0)
        sims3, m3 = _k1a(xs, cluster_centers)
        m = m3.transpose(1, 0, 2).reshape(QS, NCH)
        cids, vals = _k1c(m)
        sims2d = sims3.reshape(QS * NCH, CH)
        outs.append(_k2(sims2d, cids, vals, cluster_centers, xs))
    return jnp.concatenate(outs, axis=0)
